# baseline (speedup 1.0000x reference)
import functools

import jax
import jax.numpy as jnp
from jax import lax
from jax.experimental import pallas as pl
from jax.experimental.pallas import tpu as pltpu
from jax.experimental.pallas import tpu_sc as plsc

B, Q, W, P, D = 8, 32, 16, 64, 64
L = 16
PC = P // 4
ROWS = PC * Q


def _splat(x):
    return x if getattr(x, "shape", ()) == (L,) else jnp.full((L,), x, jnp.int32)


@functools.cache
def _build_sc_encode():
    @functools.partial(
        pl.kernel,
        mesh=plsc.VectorSubcoreMesh(core_axis_name="c", subcore_axis_name="s"),
        compiler_params=pltpu.CompilerParams(
            needs_layout_passes=False, use_tc_tiling_on_sc=False),
        out_type=jax.ShapeDtypeStruct((B * P * Q, D), jnp.float32),
        scratch_types=[
            pltpu.VMEM((Q * W,), jnp.int32),
            pltpu.VMEM((PC,), jnp.int32),
            pltpu.VMEM((Q * 4,), jnp.int32),
            pltpu.VMEM((PC * 4,), jnp.float32),
            pltpu.VMEM((Q, D), jnp.float32),
            pltpu.VMEM((Q, D), jnp.float32),
            pltpu.VMEM((PC, D), jnp.float32),
            pltpu.SemaphoreType.DMA,
            pltpu.SemaphoreType.DMA,
            pltpu.SemaphoreType.DMA,
            pltpu.SemaphoreType.DMA,
        ],
    )
    def _sc_encode(q_hbm, l_hbm, h_hbm, pr_hbm, t_hbm, out_hbm,
                   q_v, l_v, h_v, pr_v, gb_v, zb_v, lm_v,
                   sem_in, sem_g, sem_o, sem_p):
        wid = lax.axis_index("s") * 2 + lax.axis_index("c")
        b = wid // 4
        p0 = (wid % 4) * PC

        ins = [
            pltpu.async_copy(q_hbm.at[pl.ds(b * Q * W, Q * W)], q_v, sem_in),
            pltpu.async_copy(l_hbm.at[pl.ds(b * P + p0, PC)], l_v, sem_in),
            pltpu.async_copy(h_hbm.at[pl.ds(b * Q * 4, Q * 4)], h_v, sem_in),
            pltpu.async_copy(pr_hbm.at[pl.ds((b * P + p0) * 4, PC * 4)],
                             pr_v, sem_in),
        ]
        for cp in ins:
            cp.wait()

        lanes = lax.iota(jnp.int32, L)
        zero = jnp.zeros((L,), jnp.float32)
        for r in range(Q):
            for c4 in range(D // L):
                zb_v[r, pl.ds(c4 * L, L)] = zero

        prop = pr_v[pl.ds(0, L)] != 0.0
        for c4 in range(1, 4):
            prop = prop | (plsc.load_gather(pr_v, [lanes * 4 + c4]) != 0.0)
        prop_i = prop.astype(jnp.int32)

        lab = l_v[...]

        grows = [
            pltpu.async_copy(t_hbm.at[pl.ds(lab[p] * D, D)], lm_v.at[p], sem_g)
            for p in range(PC)
        ]

        hch = None
        valid_s, hsel_s, patch_s = [], [], []
        for q in range(Q):
            row = q_v[pl.ds(q * W, W)]
            if q % 4 == 0:
                hch = h_v[pl.ds((q // 4) * L, L)]
            head = hch[(q % 4) * 4]
            is_word = row != 0
            isq = plsc.all_reduce_population_count(is_word) > 0
            is_head = (row == head) & isq
            masked = jnp.where(is_head, row, 0)
            mx = jnp.max(masked)
            wi = _splat(plsc.all_reduce_ffs(masked == mx))
            wsel = plsc.load_gather(q_v, [wi + q * W])
            hsel = (wsel == head) & isq
            valid = (wsel != 0) & isq
            wsel_sc = wsel[0]
            valid_sc = valid.astype(jnp.int32)[0]
            hsel_sc = hsel.astype(jnp.int32)[0]
            valid_s.append(valid_sc)
            hsel_s.append(hsel_sc)
            patch_s.append((hsel_sc & valid_sc) != 0)
            grows.append(pltpu.async_copy(
                t_hbm.at[pl.ds(wsel_sc * D, D)], gb_v.at[q], sem_g))

        for cp in grows:
            cp.wait()

        for q in range(Q):
            @pl.when(valid_s[q] == 0)
            def _zero_row(q=q):
                for c4 in range(D // L):
                    gb_v[q, pl.ds(c4 * L, L)] = zero

        for p in range(PC):
            sp = jnp.where(prop_i[p] != 0, 1.0, 0.0)
            for c4 in range(D // L):
                lm_v[p, pl.ds(c4 * L, L)] = lm_v[p, pl.ds(c4 * L, L)] * sp

        for p in range(PC):
            dst = out_hbm.at[pl.ds(wid * ROWS + p * Q, Q)]

            @pl.when(prop_i[p] != 0)
            def _blk(dst=dst):
                pltpu.async_copy(gb_v, dst, sem_o)

            @pl.when(prop_i[p] == 0)
            def _zblk(dst=dst):
                pltpu.async_copy(zb_v, dst, sem_o)
        for p in range(PC):
            pltpu.make_async_copy(
                gb_v, out_hbm.at[pl.ds(wid * ROWS + p * Q, Q)], sem_o).wait()

        for q in range(Q):
            @pl.when(patch_s[q])
            def _patch(q=q):
                for p in range(PC):
                    pltpu.async_copy(
                        lm_v.at[p],
                        out_hbm.at[wid * ROWS + p * Q + q], sem_p)
        for q in range(Q):
            @pl.when(patch_s[q])
            def _patch_wait(q=q):
                for p in range(PC):
                    pltpu.make_async_copy(
                        lm_v.at[p],
                        out_hbm.at[wid * ROWS + p * Q + q], sem_p).wait()

    return _sc_encode


def kernel(queries, labels, heads, proposals, table):
    qf = queries.astype(jnp.int32).reshape(-1)
    lf = labels.astype(jnp.int32).reshape(-1)
    hf = heads.astype(jnp.int32).reshape(-1)
    pf = proposals.astype(jnp.float32).reshape(-1)
    tf = table.astype(jnp.float32).reshape(-1)
    out = _build_sc_encode()(qf, lf, hf, pf, tf)
    return out.reshape(B, P, Q, D)


# speedup vs baseline: 1.2909x; 1.2909x over previous
import functools

import jax
import jax.numpy as jnp
from jax import lax
from jax.experimental import pallas as pl
from jax.experimental.pallas import tpu as pltpu
from jax.experimental.pallas import tpu_sc as plsc

B, Q, W, P, D = 8, 32, 16, 64, 64
L = 16
PC = P // 4
ROWS = PC * Q


def _splat(x):
    return x if getattr(x, "shape", ()) == (L,) else jnp.full((L,), x, jnp.int32)


@functools.cache
def _build_sc_encode():
    @functools.partial(
        pl.kernel,
        mesh=plsc.VectorSubcoreMesh(core_axis_name="c", subcore_axis_name="s"),
        compiler_params=pltpu.CompilerParams(needs_layout_passes=False),
        out_type=jax.ShapeDtypeStruct((B * P * Q, D), jnp.float32),
        scratch_types=[
            pltpu.VMEM((Q * W,), jnp.int32),
            pltpu.VMEM((PC,), jnp.int32),
            pltpu.VMEM((Q * 4,), jnp.int32),
            pltpu.VMEM((PC * 4,), jnp.float32),
            pltpu.VMEM((Q, D), jnp.float32),
            pltpu.VMEM((Q, D), jnp.float32),
            pltpu.VMEM((PC, D), jnp.float32),
            pltpu.SemaphoreType.DMA,
            pltpu.SemaphoreType.DMA,
            pltpu.SemaphoreType.DMA,
            pltpu.SemaphoreType.DMA,
        ],
    )
    def _sc_encode(q_hbm, l_hbm, h_hbm, pr_hbm, t_hbm, out_hbm,
                   q_v, l_v, h_v, pr_v, gb_v, zb_v, lm_v,
                   sem_in, sem_g, sem_o, sem_p):
        wid = lax.axis_index("s") * 2 + lax.axis_index("c")
        b = wid // 4
        p0 = (wid % 4) * PC

        ins = [
            pltpu.async_copy(q_hbm.at[pl.ds(b * Q * W, Q * W)], q_v, sem_in),
            pltpu.async_copy(l_hbm.at[pl.ds(b * P + p0, PC)], l_v, sem_in),
            pltpu.async_copy(h_hbm.at[pl.ds(b * Q * 4, Q * 4)], h_v, sem_in),
            pltpu.async_copy(pr_hbm.at[pl.ds((b * P + p0) * 4, PC * 4)],
                             pr_v, sem_in),
        ]
        for cp in ins:
            cp.wait()

        lanes = lax.iota(jnp.int32, L)
        zero = jnp.zeros((L,), jnp.float32)
        for r in range(Q):
            for c4 in range(D // L):
                zb_v[r, pl.ds(c4 * L, L)] = zero

        prop = pr_v[pl.ds(0, L)] != 0.0
        for c4 in range(1, 4):
            prop = prop | (plsc.load_gather(pr_v, [lanes * 4 + c4]) != 0.0)
        prop_i = prop.astype(jnp.int32)

        lab = l_v[...]

        grows = [
            pltpu.async_copy(t_hbm.at[lab[p]], lm_v.at[p], sem_g)
            for p in range(PC)
        ]

        hch = None
        valid_s, hsel_s, patch_s = [], [], []
        for q in range(Q):
            row = q_v[pl.ds(q * W, W)]
            if q % 4 == 0:
                hch = h_v[pl.ds((q // 4) * L, L)]
            head = hch[(q % 4) * 4]
            is_word = row != 0
            isq = plsc.all_reduce_population_count(is_word) > 0
            is_head = (row == head) & isq
            masked = jnp.where(is_head, row, 0)
            mx = jnp.max(masked)
            wi = _splat(plsc.all_reduce_ffs(masked == mx))
            wsel = plsc.load_gather(q_v, [wi + q * W])
            hsel = (wsel == head) & isq
            valid = (wsel != 0) & isq
            wsel_sc = wsel[0]
            valid_sc = valid.astype(jnp.int32)[0]
            hsel_sc = hsel.astype(jnp.int32)[0]
            valid_s.append(valid_sc)
            hsel_s.append(hsel_sc)
            patch_s.append((hsel_sc & valid_sc) != 0)
            grows.append(pltpu.async_copy(
                t_hbm.at[wsel_sc], gb_v.at[q], sem_g))

        for cp in grows:
            cp.wait()

        for q in range(Q):
            @pl.when(valid_s[q] == 0)
            def _zero_row(q=q):
                for c4 in range(D // L):
                    gb_v[q, pl.ds(c4 * L, L)] = zero

        for p in range(PC):
            sp = jnp.where(prop_i[p] != 0, 1.0, 0.0)
            for c4 in range(D // L):
                lm_v[p, pl.ds(c4 * L, L)] = lm_v[p, pl.ds(c4 * L, L)] * sp

        for p in range(PC):
            dst = out_hbm.at[pl.ds(wid * ROWS + p * Q, Q)]

            @pl.when(prop_i[p] != 0)
            def _blk(dst=dst):
                pltpu.async_copy(gb_v, dst, sem_o)

            @pl.when(prop_i[p] == 0)
            def _zblk(dst=dst):
                pltpu.async_copy(zb_v, dst, sem_o)
        for p in range(PC):
            pltpu.make_async_copy(
                gb_v, out_hbm.at[pl.ds(wid * ROWS + p * Q, Q)], sem_o).wait()

        for q in range(Q):
            @pl.when(patch_s[q])
            def _patch(q=q):
                for p in range(PC):
                    pltpu.async_copy(
                        lm_v.at[p],
                        out_hbm.at[wid * ROWS + p * Q + q], sem_p)
        for q in range(Q):
            @pl.when(patch_s[q])
            def _patch_wait(q=q):
                for p in range(PC):
                    pltpu.make_async_copy(
                        lm_v.at[p],
                        out_hbm.at[wid * ROWS + p * Q + q], sem_p).wait()

    return _sc_encode


def kernel(queries, labels, heads, proposals, table):
    qf = queries.astype(jnp.int32).reshape(-1)
    lf = labels.astype(jnp.int32).reshape(-1)
    hf = heads.astype(jnp.int32).reshape(-1)
    pf = proposals.astype(jnp.float32).reshape(-1)
    out = _build_sc_encode()(qf, lf, hf, pf, table.astype(jnp.float32))
    return out.reshape(B, P, Q, D)


# speedup vs baseline: 1.5762x; 1.2210x over previous
import functools

import jax
import jax.numpy as jnp
from jax import lax
from jax.experimental import pallas as pl
from jax.experimental.pallas import tpu as pltpu
from jax.experimental.pallas import tpu_sc as plsc

B, Q, W, P, D = 8, 32, 16, 64, 64
L = 16
PC = P // 4
ROWS = PC * Q


def _splat(x):
    return x if getattr(x, "shape", ()) == (L,) else jnp.full((L,), x, jnp.int32)


@functools.cache
def _build_sc_encode():
    @functools.partial(
        pl.kernel,
        mesh=plsc.VectorSubcoreMesh(core_axis_name="c", subcore_axis_name="s"),
        compiler_params=pltpu.CompilerParams(needs_layout_passes=False),
        out_type=jax.ShapeDtypeStruct((B * P * Q, D), jnp.float32),
        scratch_types=[
            pltpu.VMEM((Q * W,), jnp.int32),
            pltpu.VMEM((PC,), jnp.int32),
            pltpu.VMEM((Q * 4,), jnp.int32),
            pltpu.VMEM((PC * 4,), jnp.float32),
            pltpu.VMEM((Q, D), jnp.float32),
            pltpu.VMEM((Q, D), jnp.float32),
            pltpu.VMEM((PC, D), jnp.float32),
            pltpu.SMEM((Q,), jnp.int32),
            pltpu.SemaphoreType.DMA,
            pltpu.SemaphoreType.DMA,
            pltpu.SemaphoreType.DMA,
            pltpu.SemaphoreType.DMA,
        ],
    )
    def _sc_encode(q_hbm, l_hbm, h_hbm, pr_hbm, t_hbm, out_hbm,
                   q_v, l_v, h_v, pr_v, gb_v, zb_v, lm_v, plist,
                   sem_in, sem_g, sem_o, sem_p):
        wid = lax.axis_index("s") * 2 + lax.axis_index("c")
        b = wid // 4
        p0 = (wid % 4) * PC

        ins = [
            pltpu.async_copy(q_hbm.at[pl.ds(b * Q * W, Q * W)], q_v, sem_in),
            pltpu.async_copy(l_hbm.at[pl.ds(b * P + p0, PC)], l_v, sem_in),
            pltpu.async_copy(h_hbm.at[pl.ds(b * Q * 4, Q * 4)], h_v, sem_in),
            pltpu.async_copy(pr_hbm.at[pl.ds((b * P + p0) * 4, PC * 4)],
                             pr_v, sem_in),
        ]
        for cp in ins:
            cp.wait()

        lanes = lax.iota(jnp.int32, L)
        zero = jnp.zeros((L,), jnp.float32)
        for r in range(Q):
            for c4 in range(D // L):
                zb_v[r, pl.ds(c4 * L, L)] = zero

        prop = pr_v[pl.ds(0, L)] != 0.0
        for c4 in range(1, 4):
            prop = prop | (plsc.load_gather(pr_v, [lanes * 4 + c4]) != 0.0)
        prop_i = prop.astype(jnp.int32)

        lab = l_v[...]

        grows = [
            pltpu.async_copy(t_hbm.at[lab[p]], lm_v.at[p], sem_g)
            for p in range(PC)
        ]

        valid_s = []
        npatch = jnp.int32(0)
        for qc in range(Q // L):
            qbase = qc * L
            headv = plsc.load_gather(h_v, [(qbase + lanes) * 4])
            col0 = plsc.load_gather(q_v, [(qbase + lanes) * W])
            qmax = col0
            hmax = jnp.where(col0 == headv, col0, 0)
            for w in range(1, W):
                col = plsc.load_gather(q_v, [(qbase + lanes) * W + w])
                qmax = jnp.maximum(qmax, col)
                hmax = jnp.maximum(hmax, jnp.where(col == headv, col, 0))
            isq = qmax > 0
            wselv = jnp.where(hmax > 0, hmax, col0)
            hselv = ((wselv == headv) & isq).astype(jnp.int32)
            validv = ((wselv != 0) & isq).astype(jnp.int32)
            for i in range(L):
                q = qbase + i
                valid_sc = validv[i]
                valid_s.append(valid_sc)
                plist[npatch] = jnp.int32(q)
                npatch = npatch + (hselv[i] & valid_sc)
                grows.append(pltpu.async_copy(
                    t_hbm.at[wselv[i]], gb_v.at[q], sem_g))

        for cp in grows:
            cp.wait()

        for q in range(Q):
            @pl.when(valid_s[q] == 0)
            def _zero_row(q=q):
                for c4 in range(D // L):
                    gb_v[q, pl.ds(c4 * L, L)] = zero

        for p in range(PC):
            sp = jnp.where(prop_i[p] != 0, 1.0, 0.0)
            for c4 in range(D // L):
                lm_v[p, pl.ds(c4 * L, L)] = lm_v[p, pl.ds(c4 * L, L)] * sp

        for p in range(PC):
            dst = out_hbm.at[pl.ds(wid * ROWS + p * Q, Q)]

            @pl.when(prop_i[p] != 0)
            def _blk(dst=dst):
                pltpu.async_copy(gb_v, dst, sem_o)

            @pl.when(prop_i[p] == 0)
            def _zblk(dst=dst):
                pltpu.async_copy(zb_v, dst, sem_o)
        for p in range(PC):
            pltpu.make_async_copy(
                gb_v, out_hbm.at[pl.ds(wid * ROWS + p * Q, Q)], sem_o).wait()

        def _patch(i, carry):
            qq = plist[i]
            for p in range(PC):
                pltpu.async_copy(
                    lm_v.at[p], out_hbm.at[wid * ROWS + p * Q + qq], sem_p)
            return carry
        lax.fori_loop(0, npatch, _patch, 0)

        def _patch_wait(i, carry):
            pltpu.make_async_copy(t_hbm.at[0], lm_v.at[0], sem_p).wait()
            return carry
        lax.fori_loop(0, npatch * PC, _patch_wait, 0)

    return _sc_encode


def kernel(queries, labels, heads, proposals, table):
    qf = queries.astype(jnp.int32).reshape(-1)
    lf = labels.astype(jnp.int32).reshape(-1)
    hf = heads.astype(jnp.int32).reshape(-1)
    pf = proposals.astype(jnp.float32).reshape(-1)
    out = _build_sc_encode()(qf, lf, hf, pf, table.astype(jnp.float32))
    return out.reshape(B, P, Q, D)


# speedup vs baseline: 1.7860x; 1.1331x over previous
import functools

import jax
import jax.numpy as jnp
from jax import lax
from jax.experimental import pallas as pl
from jax.experimental.pallas import tpu as pltpu
from jax.experimental.pallas import tpu_sc as plsc

B, Q, W, P, D = 8, 32, 16, 64, 64
L = 16
PC = P // 4
ROWS = PC * Q


@functools.cache
def _build_sc_encode():
    @functools.partial(
        pl.kernel,
        mesh=plsc.VectorSubcoreMesh(core_axis_name="c", subcore_axis_name="s"),
        compiler_params=pltpu.CompilerParams(needs_layout_passes=False),
        out_type=jax.ShapeDtypeStruct((B * P * Q, D), jnp.float32),
        scratch_types=[
            pltpu.VMEM((W, Q), jnp.int32),
            pltpu.VMEM((PC,), jnp.int32),
            pltpu.VMEM((Q,), jnp.int32),
            pltpu.VMEM((4, PC), jnp.float32),
            pltpu.VMEM((Q, D), jnp.float32),
            pltpu.VMEM((Q, D), jnp.float32),
            pltpu.VMEM((PC, D), jnp.float32),
            pltpu.SMEM((Q,), jnp.int32),
            pltpu.SemaphoreType.DMA,
            pltpu.SemaphoreType.DMA,
            pltpu.SemaphoreType.DMA,
            pltpu.SemaphoreType.DMA,
        ],
    )
    def _sc_encode(q_hbm, l_hbm, h_hbm, pr_hbm, t_hbm, out_hbm,
                   q_v, l_v, h_v, pr_v, gb_v, zb_v, lm_v, plist,
                   sem_in, sem_g, sem_o, sem_p):
        wid = lax.axis_index("s") * 2 + lax.axis_index("c")
        b = wid // 4
        p0 = (wid % 4) * PC

        ins = [
            pltpu.async_copy(q_hbm.at[b], q_v, sem_in),
            pltpu.async_copy(l_hbm.at[pl.ds(b * P + p0, PC)], l_v, sem_in),
            pltpu.async_copy(h_hbm.at[b, 0], h_v, sem_in),
        ] + [
            pltpu.async_copy(pr_hbm.at[b, c, pl.ds(p0, PC)], pr_v.at[c],
                             sem_in)
            for c in range(4)
        ]
        for cp in ins:
            cp.wait()

        lanes = lax.iota(jnp.int32, L)
        zero = jnp.zeros((L,), jnp.float32)
        for r in range(Q):
            for c4 in range(D // L):
                zb_v[r, pl.ds(c4 * L, L)] = zero

        prop = pr_v[0, ...] != 0.0
        for c4 in range(1, 4):
            prop = prop | (pr_v[c4, ...] != 0.0)
        prop_i = prop.astype(jnp.int32)

        lab = l_v[...]

        grows = [
            pltpu.async_copy(t_hbm.at[lab[p]], lm_v.at[p], sem_g)
            for p in range(PC)
        ]

        valid_s = []
        npatch = jnp.int32(0)
        for qc in range(Q // L):
            qbase = qc * L
            headv = h_v[pl.ds(qbase, L)]
            col0 = q_v[0, pl.ds(qbase, L)]
            qmax = col0
            hmax = jnp.where(col0 == headv, col0, 0)
            for w in range(1, W):
                col = q_v[w, pl.ds(qbase, L)]
                qmax = jnp.maximum(qmax, col)
                hmax = jnp.maximum(hmax, jnp.where(col == headv, col, 0))
            isq = qmax > 0
            wselv = jnp.where(hmax > 0, hmax, col0)
            hselv = ((wselv == headv) & isq).astype(jnp.int32)
            validv = ((wselv != 0) & isq).astype(jnp.int32)
            for i in range(L):
                q = qbase + i
                valid_sc = validv[i]
                valid_s.append(valid_sc)
                plist[npatch] = jnp.int32(q)
                npatch = npatch + (hselv[i] & valid_sc)
                grows.append(pltpu.async_copy(
                    t_hbm.at[wselv[i]], gb_v.at[q], sem_g))

        for cp in grows:
            cp.wait()

        for q in range(Q):
            @pl.when(valid_s[q] == 0)
            def _zero_row(q=q):
                for c4 in range(D // L):
                    gb_v[q, pl.ds(c4 * L, L)] = zero

        for p in range(PC):
            sp = jnp.where(prop_i[p] != 0, 1.0, 0.0)
            for c4 in range(D // L):
                lm_v[p, pl.ds(c4 * L, L)] = lm_v[p, pl.ds(c4 * L, L)] * sp

        for p in range(PC):
            dst = out_hbm.at[pl.ds(wid * ROWS + p * Q, Q)]

            @pl.when(prop_i[p] != 0)
            def _blk(dst=dst):
                pltpu.async_copy(gb_v, dst, sem_o)

            @pl.when(prop_i[p] == 0)
            def _zblk(dst=dst):
                pltpu.async_copy(zb_v, dst, sem_o)
        for p in range(PC):
            pltpu.make_async_copy(
                gb_v, out_hbm.at[pl.ds(wid * ROWS + p * Q, Q)], sem_o).wait()

        def _patch(i, carry):
            qq = plist[i]
            for p in range(PC):
                pltpu.async_copy(
                    lm_v.at[p], out_hbm.at[wid * ROWS + p * Q + qq], sem_p)
            return carry
        lax.fori_loop(0, npatch, _patch, 0)

        def _patch_wait(i, carry):
            pltpu.make_async_copy(t_hbm.at[0], lm_v.at[0], sem_p).wait()
            return carry
        lax.fori_loop(0, npatch * PC, _patch_wait, 0)

    return _sc_encode


def kernel(queries, labels, heads, proposals, table):
    qt = jnp.swapaxes(queries.astype(jnp.int32), 1, 2)
    ht = jnp.swapaxes(heads.astype(jnp.int32), 1, 2)
    pt = jnp.swapaxes(proposals.astype(jnp.float32), 1, 2)
    lf = labels.astype(jnp.int32).reshape(-1)
    out = _build_sc_encode()(qt, lf, ht, pt, table.astype(jnp.float32))
    return out.reshape(B, P, Q, D)
